# Initial kernel scaffold; baseline (speedup 1.0000x reference)
import jax
import jax.numpy as jnp
from jax.experimental import pallas as pl


def kernel(x, edge_index, edge_attr, W, b, alphas_param):
    raise NotImplementedError("write your pallas kernel here")


# speedup vs baseline: 5.0196x; 5.0196x over previous
import jax
import jax.numpy as jnp
from jax import lax
from jax.experimental import pallas as pl
from jax.experimental.pallas import tpu as pltpu
from jax.experimental.pallas import tpu_sc as plsc

NN = 10000
NNP = 10240
NE = 160000
D = 256
H = 128
NC = 2
NS = 16
EPT = NE // NS
CHA = 80
NCHA = EPT // CHA
CHB = 16
NCHB = EPT // CHB
RT = NNP // NS
RC = 16
NRC = RT // RC
MMB = 640
DEPTH = 3


def _mm_body(x_ref, w_ref, b_ref, o_ref):
    c = pl.program_id(0)
    bias = b_ref[pl.ds(c, 1), :]
    o_ref[...] = (
        jnp.dot(x_ref[...], w_ref[...], preferred_element_type=jnp.float32)
        + bias
    )


def _linear(xp, W, b2):
    return pl.pallas_call(
        _mm_body,
        grid=(NC, NNP // MMB),
        in_specs=[
            pl.BlockSpec((MMB, D), lambda c, k: (k, 0)),
            pl.BlockSpec((D, H), lambda c, k: (0, c)),
            pl.BlockSpec((NC, H), lambda c, k: (0, 0)),
        ],
        out_specs=pl.BlockSpec((MMB, H), lambda c, k: (c * (NNP // MMB) + k, 0)),
        out_shape=jax.ShapeDtypeStruct((NC * NNP, H), jnp.float32),
    )(xp, W, b2)


def _prep_body(srcs, dsts, eas,
               ew_out,
               deg,
               src2d, dst2d, ew2d, z1d, dinv_v,
               dsem):
    c = lax.axis_index("c")
    s = lax.axis_index("s")
    z16 = jnp.zeros((16,), jnp.float32)

    pltpu.sync_copy(srcs.at[s], src2d)
    pltpu.sync_copy(dsts.at[s], dst2d)
    pltpu.sync_copy(eas.at[s], ew2d)

    @pl.loop(0, RT // 16)
    def _(i):
        z1d[pl.ds(i * 16, 16)] = z16

    pltpu.sync_copy(z1d, deg.at[pl.ds(s * RT, RT)])
    plsc.subcore_barrier()

    def deg_wait(ch):
        pltpu.make_async_copy(eas.at[s, ch], ew2d.at[ch], dsem).wait()

    @pl.loop(0, NCHA)
    def _(ch):
        pltpu.async_copy(ew2d.at[ch], deg.at[dst2d.at[ch]], dsem, add=True)

        @pl.when(ch >= 8)
        def _():
            deg_wait(ch - 8)

    @pl.loop(NCHA - 8, NCHA)
    def _(ch):
        deg_wait(ch)

    plsc.subcore_barrier()

    pltpu.sync_copy(deg.at[pl.ds(s * RT, RT)], z1d)

    @pl.loop(0, RT // 16)
    def _(i):
        sl = pl.ds(i * 16, 16)
        v = z1d[sl]
        vm = jnp.maximum(v, 1e-12)
        bits = lax.bitcast_convert_type(vm, jnp.int32)
        y = lax.bitcast_convert_type(
            jnp.int32(0x5F3759DF) - (bits >> 1), jnp.float32
        )
        for _ in range(3):
            y = y * (1.5 - 0.5 * vm * y * y)
        z1d[sl] = jnp.where(v > 0.0, y, 0.0)

    pltpu.sync_copy(z1d, deg.at[pl.ds(s * RT, RT)])
    plsc.subcore_barrier()
    pltpu.sync_copy(deg, dinv_v)

    @pl.loop(0, NCHA)
    def _(ch):
        for k in range(CHA // 16):
            sl = pl.ds(k * 16, 16)
            dv_s = plsc.load_gather(dinv_v, [src2d[ch, sl]])
            dv_d = plsc.load_gather(dinv_v, [dst2d[ch, sl]])
            ew2d[ch, sl] = dv_s * ew2d[ch, sl] * dv_d

    @pl.when(c == 0)
    def _():
        pltpu.sync_copy(ew2d.at[pl.ds(0, 63)], ew_out.at[s, pl.ds(0, 63)])

    @pl.when(c == 1)
    def _():
        pltpu.sync_copy(ew2d.at[pl.ds(63, 62)], ew_out.at[s, pl.ds(63, 62)])


_prep_call = pl.kernel(
    _prep_body,
    out_type=[jax.ShapeDtypeStruct((NS, NCHA, CHA), jnp.float32)],
    mesh=plsc.VectorSubcoreMesh(
        core_axis_name="c", subcore_axis_name="s", num_cores=NC, num_subcores=NS
    ),
    compiler_params=pltpu.CompilerParams(
        needs_layout_passes=False, use_tc_tiling_on_sc=False
    ),
    scratch_types=[
        pltpu.VMEM_SHARED((NNP,), jnp.float32),
        pltpu.VMEM((NCHA, CHA), jnp.int32),
        pltpu.VMEM((NCHA, CHA), jnp.int32),
        pltpu.VMEM((NCHA, CHA), jnp.float32),
        pltpu.VMEM((RT,), jnp.float32),
        pltpu.VMEM((NNP,), jnp.float32),
        pltpu.SemaphoreType.DMA,
    ],
)


def _spmm_body(xsrc, srcs, dsts, ews, coefs,
               y1, y2, y3,
               acc,
               src2d, dst2d, ew2d, g0, g1, s0, s1, x20, coefv,
               gs0, gs1, ss0, ss1):
    c = lax.axis_index("c")
    s = lax.axis_index("s")
    coff = c * NNP

    z16 = jnp.zeros((16,), jnp.float32)

    pltpu.sync_copy(srcs.at[s], src2d)
    pltpu.sync_copy(dsts.at[s], dst2d)
    pltpu.sync_copy(ews.at[s], ew2d)
    pltpu.sync_copy(coefs, coefv)

    @pl.loop(0, NCHB)
    def _(ch):
        src2d[ch, :] = src2d[ch, :] + coff

    levels = (
        (xsrc, xsrc, xsrc, y1, 0),
        (y1, y1, xsrc, y2, 1),
        (y2, y2, y1, y3, 2),
    )
    for srcref, xlref, x2ref, yout, li in levels:
        @pl.loop(0, CHB)
        def _(r):
            for j in range(H // 16):
                s0[r, pl.ds(j * 16, 16)] = z16

        @pl.loop(0, NRC)
        def _(k):
            pltpu.sync_copy(s0, acc.at[pl.ds(s * RT + k * RC, RC)])

        plsc.subcore_barrier()

        def scat_wait(sb, ssem):
            pltpu.make_async_copy(xsrc.at[pl.ds(0, CHB)], sb, ssem).wait()

        pltpu.async_copy(srcref.at[src2d.at[0]], g0, gs0)
        pltpu.async_copy(srcref.at[src2d.at[1]], g1, gs1)

        @pl.loop(0, NCHB)
        def _(ch):
            for b, (gb, sb, gsem, ssem) in enumerate(
                ((g0, s0, gs0, ss0), (g1, s1, gs1, ss1))
            ):
                @pl.when(ch % 2 == b)
                def _():
                    pltpu.make_async_copy(srcref.at[src2d.at[ch]], gb, gsem).wait()

                    @pl.when(ch >= 2)
                    def _():
                        scat_wait(sb, ssem)

                    wv = ew2d[ch, :]
                    for e in range(CHB):
                        w = wv[e]
                        for j in range(H // 16):
                            sl = pl.ds(j * 16, 16)
                            sb[e, sl] = gb[e, sl] * w

                    @pl.when(ch + 2 < NCHB)
                    def _():
                        pltpu.async_copy(srcref.at[src2d.at[ch + 2]], gb, gsem)

                    pltpu.async_copy(sb, acc.at[dst2d.at[ch]], ssem, add=True)

        scat_wait(s1, ss1)
        scat_wait(s0, ss0)
        plsc.subcore_barrier()

        cv = coefv[...]
        cY = cv[3 * li + 0]
        cX = cv[3 * li + 1]
        cX2 = cv[3 * li + 2]

        @pl.loop(0, NRC)
        def _(k):
            base = s * RT + k * RC
            pltpu.sync_copy(acc.at[pl.ds(base, RC)], g0)
            pltpu.sync_copy(xlref.at[pl.ds(coff + base, RC)], g1)
            pltpu.sync_copy(x2ref.at[pl.ds(coff + base, RC)], x20)

            @pl.loop(0, RC)
            def _(r):
                for j in range(H // 16):
                    sl = pl.ds(j * 16, 16)
                    g0[r, sl] = cY * g0[r, sl] + cX * g1[r, sl] + cX2 * x20[r, sl]

            pltpu.sync_copy(g0, yout.at[pl.ds(coff + base, RC)])

        plsc.subcore_barrier()


_spmm_call = pl.kernel(
    _spmm_body,
    out_type=[jax.ShapeDtypeStruct((NC * NNP, H), jnp.float32)] * DEPTH,
    mesh=plsc.VectorSubcoreMesh(
        core_axis_name="c", subcore_axis_name="s", num_cores=NC, num_subcores=NS
    ),
    compiler_params=pltpu.CompilerParams(
        needs_layout_passes=False, use_tc_tiling_on_sc=False
    ),
    scratch_types=[
        pltpu.VMEM_SHARED((NNP, H), jnp.float32),
        pltpu.VMEM((NCHB, CHB), jnp.int32),
        pltpu.VMEM((NCHB, CHB), jnp.int32),
        pltpu.VMEM((NCHB, CHB), jnp.float32),
        pltpu.VMEM((CHB, H), jnp.float32),
        pltpu.VMEM((CHB, H), jnp.float32),
        pltpu.VMEM((CHB, H), jnp.float32),
        pltpu.VMEM((CHB, H), jnp.float32),
        pltpu.VMEM((RC, H), jnp.float32),
        pltpu.VMEM((16,), jnp.float32),
        pltpu.SemaphoreType.DMA,
        pltpu.SemaphoreType.DMA,
        pltpu.SemaphoreType.DMA,
        pltpu.SemaphoreType.DMA,
    ],
)


def _jacobi_coefs(alphas_param):
    alphas = jnp.tanh(alphas_param)
    a_, b_, l_, r_ = 1.0, 1.0, -1.0, 1.0
    cs = []
    coef1 = ((a_ - b_) / 2.0 - (a_ + b_ + 2.0) / 2.0 * (l_ + r_) / (r_ - l_))
    cs += [((a_ + b_ + 2.0) / (r_ - l_)) * alphas[0], coef1 * alphas[0],
           jnp.float32(0.0)]
    for L in range(2, DEPTH + 1):
        coef_l = 2.0 * L * (L + a_ + b_) * (2.0 * L - 2.0 + a_ + b_)
        coef_lm1_1 = (2.0 * L + a_ + b_ - 1.0) * (2.0 * L + a_ + b_) * (
            2.0 * L + a_ + b_ - 2.0)
        coef_lm1_2 = (2.0 * L + a_ + b_ - 1.0) * (a_ * a_ - b_ * b_)
        coef_lm2 = 2.0 * (L - 1.0 + a_) * (L - 1.0 + b_) * (2.0 * L + a_ + b_)
        tmp1 = alphas[L - 1] * (coef_lm1_1 / coef_l)
        tmp2 = alphas[L - 1] * (coef_lm1_2 / coef_l)
        tmp3 = alphas[L - 1] * alphas[L - 2] * (coef_lm2 / coef_l)
        tmp1_2 = tmp1 * (2.0 / (r_ - l_))
        tmp2_2 = tmp1 * ((r_ + l_) / (r_ - l_)) + tmp2
        cs += [tmp1_2, -tmp2_2, -tmp3]
    cs += [jnp.float32(0.0)] * (16 - len(cs))
    return jnp.stack(cs).astype(jnp.float32)


def kernel(x, edge_index, edge_attr, W, b, alphas_param):
    ei = edge_index.astype(jnp.int32)
    ea = edge_attr.astype(jnp.float32)
    srcs_a = ei[0].reshape(NS, NCHA, CHA)
    dsts_a = ei[1].reshape(NS, NCHA, CHA)
    eas_a = ea.reshape(NS, NCHA, CHA)
    srcs_b = ei[0].reshape(NS, NCHB, CHB)
    dsts_b = ei[1].reshape(NS, NCHB, CHB)
    xp = jnp.pad(x, ((0, NNP - NN), (0, 0)))
    b2 = b.reshape(NC, H)

    xsrc = _linear(xp, W, b2)
    coefs = _jacobi_coefs(alphas_param)
    (ew,) = _prep_call(srcs_a, dsts_a, eas_a)
    ews_b = ew.reshape(NS, NCHB, CHB)
    y1, y2, y3 = _spmm_call(xsrc, srcs_b, dsts_b, ews_b, coefs)

    def half(yv):
        return jnp.concatenate([yv[:NN], yv[NNP:NNP + NN]], axis=1)

    return jnp.stack([half(xsrc), half(y1), half(y2), half(y3)], axis=1)


# speedup vs baseline: 7.1197x; 1.4184x over previous
import jax
import jax.numpy as jnp
from jax import lax
from jax.experimental import pallas as pl
from jax.experimental.pallas import tpu as pltpu
from jax.experimental.pallas import tpu_sc as plsc

NN = 10000
NNP = 10240
NE = 160000
D = 256
H = 128
NC = 2
NS = 16
EPT = NE // NS
CHA = 80
NCHA = EPT // CHA
CHB = 16
NCHB = EPT // CHB
RT = NNP // NS
RC = 16
NRC = RT // RC
MMB = 640
DEPTH = 3


def _mm_body(x_ref, w_ref, b_ref, o_ref):
    c = pl.program_id(0)
    bias = b_ref[pl.ds(c, 1), :]
    o_ref[...] = (
        jnp.dot(x_ref[...], w_ref[...], preferred_element_type=jnp.float32)
        + bias
    )


def _linear(xp, W, b2):
    return pl.pallas_call(
        _mm_body,
        grid=(NC, NNP // MMB),
        in_specs=[
            pl.BlockSpec((MMB, D), lambda c, k: (k, 0)),
            pl.BlockSpec((D, H), lambda c, k: (0, c)),
            pl.BlockSpec((NC, H), lambda c, k: (0, 0)),
        ],
        out_specs=pl.BlockSpec((MMB, H), lambda c, k: (c * (NNP // MMB) + k, 0)),
        out_shape=jax.ShapeDtypeStruct((NC * NNP, H), jnp.float32),
    )(xp, W, b2)


def _prep_body(srcs, dsts, eas,
               ew_out,
               deg,
               src2d, dst2d, ew2d, z1d, dinv_v,
               dsem):
    c = lax.axis_index("c")
    s = lax.axis_index("s")
    z16 = jnp.zeros((16,), jnp.float32)

    pltpu.sync_copy(srcs.at[s], src2d)
    pltpu.sync_copy(dsts.at[s], dst2d)
    pltpu.sync_copy(eas.at[s], ew2d)

    @pl.loop(0, RT // 16)
    def _(i):
        z1d[pl.ds(i * 16, 16)] = z16

    pltpu.sync_copy(z1d, deg.at[pl.ds(s * RT, RT)])
    plsc.subcore_barrier()

    def deg_wait(ch):
        pltpu.make_async_copy(eas.at[s, ch], ew2d.at[ch], dsem).wait()

    @pl.loop(0, NCHA)
    def _(ch):
        pltpu.async_copy(ew2d.at[ch], deg.at[dst2d.at[ch]], dsem, add=True)

        @pl.when(ch >= 8)
        def _():
            deg_wait(ch - 8)

    @pl.loop(NCHA - 8, NCHA)
    def _(ch):
        deg_wait(ch)

    plsc.subcore_barrier()

    pltpu.sync_copy(deg.at[pl.ds(s * RT, RT)], z1d)

    @pl.loop(0, RT // 16)
    def _(i):
        sl = pl.ds(i * 16, 16)
        v = z1d[sl]
        vm = jnp.maximum(v, 1e-12)
        bits = lax.bitcast_convert_type(vm, jnp.int32)
        y = lax.bitcast_convert_type(
            jnp.int32(0x5F3759DF) - (bits >> 1), jnp.float32
        )
        for _ in range(3):
            y = y * (1.5 - 0.5 * vm * y * y)
        z1d[sl] = jnp.where(v > 0.0, y, 0.0)

    pltpu.sync_copy(z1d, deg.at[pl.ds(s * RT, RT)])
    plsc.subcore_barrier()
    pltpu.sync_copy(deg, dinv_v)

    @pl.loop(0, NCHA)
    def _(ch):
        for k in range(CHA // 16):
            sl = pl.ds(k * 16, 16)
            dv_s = plsc.load_gather(dinv_v, [src2d[ch, sl]])
            dv_d = plsc.load_gather(dinv_v, [dst2d[ch, sl]])
            ew2d[ch, sl] = dv_s * ew2d[ch, sl] * dv_d

    @pl.when(c == 0)
    def _():
        pltpu.sync_copy(ew2d.at[pl.ds(0, 63)], ew_out.at[s, pl.ds(0, 63)])

    @pl.when(c == 1)
    def _():
        pltpu.sync_copy(ew2d.at[pl.ds(63, 62)], ew_out.at[s, pl.ds(63, 62)])


_prep_call = pl.kernel(
    _prep_body,
    out_type=[jax.ShapeDtypeStruct((NS, NCHA, CHA), jnp.float32)],
    mesh=plsc.VectorSubcoreMesh(
        core_axis_name="c", subcore_axis_name="s", num_cores=NC, num_subcores=NS
    ),
    compiler_params=pltpu.CompilerParams(
        needs_layout_passes=False, use_tc_tiling_on_sc=False
    ),
    scratch_types=[
        pltpu.VMEM_SHARED((NNP,), jnp.float32),
        pltpu.VMEM((NCHA, CHA), jnp.int32),
        pltpu.VMEM((NCHA, CHA), jnp.int32),
        pltpu.VMEM((NCHA, CHA), jnp.float32),
        pltpu.VMEM((RT,), jnp.float32),
        pltpu.VMEM((NNP,), jnp.float32),
        pltpu.SemaphoreType.DMA,
    ],
)


def _spmm_body(xsrc, srcs, dsts, ews, coefs,
               y1, y2, y3,
               acc,
               src2d, dst2d, ew2d, g0, g1, g2, g3, s0, s1, s2, s3, coefv,
               gs0, gs1, gs2, gs3, ss0, ss1, ss2, ss3, cs0, cs1, os0, os1):
    c = lax.axis_index("c")
    s = lax.axis_index("s")
    coff = c * NNP

    z16 = jnp.zeros((16,), jnp.float32)

    pltpu.sync_copy(srcs.at[s], src2d)
    pltpu.sync_copy(dsts.at[s], dst2d)
    pltpu.sync_copy(ews.at[s], ew2d)
    pltpu.sync_copy(coefs, coefv)

    @pl.loop(0, NCHB)
    def _(ch):
        src2d[ch, :] = src2d[ch, :] + coff

    levels = (
        (xsrc, xsrc, xsrc, y1, 0),
        (y1, y1, xsrc, y2, 1),
        (y2, y2, y1, y3, 2),
    )
    def dummy_wait(dst, sem):
        pltpu.make_async_copy(xsrc.at[pl.ds(0, CHB)], dst, sem).wait()

    for srcref, xlref, x2ref, yout, li in levels:
        @pl.loop(0, CHB)
        def _(r):
            for j in range(H // 16):
                s0[r, pl.ds(j * 16, 16)] = z16

        @pl.loop(0, NRC)
        def _(k):
            pltpu.sync_copy(s0, acc.at[pl.ds(s * RT + k * RC, RC)])

        plsc.subcore_barrier()

        quads = (
            (g0, s0, gs0, ss0), (g1, s1, gs1, ss1),
            (g2, s2, gs2, ss2), (g3, s3, gs3, ss3),
        )
        for b in range(4):
            pltpu.async_copy(srcref.at[src2d.at[b]], quads[b][0], quads[b][2])

        @pl.loop(0, NCHB)
        def _(ch):
            for b, (gb, sb, gsem, ssem) in enumerate(quads):
                @pl.when(ch % 4 == b)
                def _():
                    pltpu.make_async_copy(srcref.at[src2d.at[ch]], gb, gsem).wait()

                    @pl.when(ch >= 4)
                    def _():
                        dummy_wait(sb, ssem)

                    wv = ew2d[ch, :]
                    for e in range(CHB):
                        w = wv[e]
                        for j in range(H // 16):
                            sl = pl.ds(j * 16, 16)
                            sb[e, sl] = gb[e, sl] * w

                    @pl.when(ch + 4 < NCHB)
                    def _():
                        pltpu.async_copy(srcref.at[src2d.at[ch + 4]], gb, gsem)

                    pltpu.async_copy(sb, acc.at[dst2d.at[ch]], ssem, add=True)

        for b in range(4):
            dummy_wait(quads[b][1], quads[b][3])
        plsc.subcore_barrier()

        cv = coefv[...]
        cY = cv[3 * li + 0]
        cX = cv[3 * li + 1]
        cX2 = cv[3 * li + 2]

        @pl.loop(0, NRC)
        def _(k):
            base = s * RT + k * RC
            pltpu.sync_copy(acc.at[pl.ds(base, RC)], g0)
            pltpu.sync_copy(xlref.at[pl.ds(coff + base, RC)], g1)
            pltpu.sync_copy(x2ref.at[pl.ds(coff + base, RC)], s0)

            @pl.loop(0, RC)
            def _(r):
                for j in range(H // 16):
                    sl = pl.ds(j * 16, 16)
                    g0[r, sl] = cY * g0[r, sl] + cX * g1[r, sl] + cX2 * s0[r, sl]

            pltpu.sync_copy(g0, yout.at[pl.ds(coff + base, RC)])

        plsc.subcore_barrier()


_spmm_call = pl.kernel(
    _spmm_body,
    out_type=[jax.ShapeDtypeStruct((NC * NNP, H), jnp.float32)] * DEPTH,
    mesh=plsc.VectorSubcoreMesh(
        core_axis_name="c", subcore_axis_name="s", num_cores=NC, num_subcores=NS
    ),
    compiler_params=pltpu.CompilerParams(
        needs_layout_passes=False, use_tc_tiling_on_sc=False
    ),
    scratch_types=[
        pltpu.VMEM_SHARED((NNP, H), jnp.float32),
        pltpu.VMEM((NCHB, CHB), jnp.int32),
        pltpu.VMEM((NCHB, CHB), jnp.int32),
        pltpu.VMEM((NCHB, CHB), jnp.float32),
        pltpu.VMEM((CHB, H), jnp.float32),
        pltpu.VMEM((CHB, H), jnp.float32),
        pltpu.VMEM((CHB, H), jnp.float32),
        pltpu.VMEM((CHB, H), jnp.float32),
        pltpu.VMEM((CHB, H), jnp.float32),
        pltpu.VMEM((CHB, H), jnp.float32),
        pltpu.VMEM((CHB, H), jnp.float32),
        pltpu.VMEM((CHB, H), jnp.float32),
        pltpu.VMEM((16,), jnp.float32),
        pltpu.SemaphoreType.DMA,
        pltpu.SemaphoreType.DMA,
        pltpu.SemaphoreType.DMA,
        pltpu.SemaphoreType.DMA,
        pltpu.SemaphoreType.DMA,
        pltpu.SemaphoreType.DMA,
        pltpu.SemaphoreType.DMA,
        pltpu.SemaphoreType.DMA,
        pltpu.SemaphoreType.DMA,
        pltpu.SemaphoreType.DMA,
        pltpu.SemaphoreType.DMA,
        pltpu.SemaphoreType.DMA,
    ],
)


def _jacobi_coefs(alphas_param):
    alphas = jnp.tanh(alphas_param)
    a_, b_, l_, r_ = 1.0, 1.0, -1.0, 1.0
    cs = []
    coef1 = ((a_ - b_) / 2.0 - (a_ + b_ + 2.0) / 2.0 * (l_ + r_) / (r_ - l_))
    cs += [((a_ + b_ + 2.0) / (r_ - l_)) * alphas[0], coef1 * alphas[0],
           jnp.float32(0.0)]
    for L in range(2, DEPTH + 1):
        coef_l = 2.0 * L * (L + a_ + b_) * (2.0 * L - 2.0 + a_ + b_)
        coef_lm1_1 = (2.0 * L + a_ + b_ - 1.0) * (2.0 * L + a_ + b_) * (
            2.0 * L + a_ + b_ - 2.0)
        coef_lm1_2 = (2.0 * L + a_ + b_ - 1.0) * (a_ * a_ - b_ * b_)
        coef_lm2 = 2.0 * (L - 1.0 + a_) * (L - 1.0 + b_) * (2.0 * L + a_ + b_)
        tmp1 = alphas[L - 1] * (coef_lm1_1 / coef_l)
        tmp2 = alphas[L - 1] * (coef_lm1_2 / coef_l)
        tmp3 = alphas[L - 1] * alphas[L - 2] * (coef_lm2 / coef_l)
        tmp1_2 = tmp1 * (2.0 / (r_ - l_))
        tmp2_2 = tmp1 * ((r_ + l_) / (r_ - l_)) + tmp2
        cs += [tmp1_2, -tmp2_2, -tmp3]
    cs += [jnp.float32(0.0)] * (16 - len(cs))
    return jnp.stack(cs).astype(jnp.float32)


def kernel(x, edge_index, edge_attr, W, b, alphas_param):
    ei = edge_index.astype(jnp.int32)
    ea = edge_attr.astype(jnp.float32)
    srcs_a = ei[0].reshape(NS, NCHA, CHA)
    dsts_a = ei[1].reshape(NS, NCHA, CHA)
    eas_a = ea.reshape(NS, NCHA, CHA)
    srcs_b = ei[0].reshape(NS, NCHB, CHB)
    dsts_b = ei[1].reshape(NS, NCHB, CHB)
    xp = jnp.pad(x, ((0, NNP - NN), (0, 0)))
    b2 = b.reshape(NC, H)

    xsrc = _linear(xp, W, b2)
    coefs = _jacobi_coefs(alphas_param)
    (ew,) = _prep_call(srcs_a, dsts_a, eas_a)
    ews_b = ew.reshape(NS, NCHB, CHB)
    y1, y2, y3 = _spmm_call(xsrc, srcs_b, dsts_b, ews_b, coefs)

    def half(yv):
        return jnp.concatenate([yv[:NN], yv[NNP:NNP + NN]], axis=1)

    return jnp.stack([half(xsrc), half(y1), half(y2), half(y3)], axis=1)


# speedup vs baseline: 8.3723x; 1.1759x over previous
import jax
import jax.numpy as jnp
from jax import lax
from jax.experimental import pallas as pl
from jax.experimental.pallas import tpu as pltpu
from jax.experimental.pallas import tpu_sc as plsc

NN = 10000
NNP = 10240
NE = 160000
D = 256
H = 128
NC = 2
NS = 16
EPT = NE // NS
CHA = 80
NCHA = EPT // CHA
CHB = 16
NCHB = EPT // CHB
RT = NNP // NS
RC = 16
NRC = RT // RC
MMB = 640
DEPTH = 3


def _mm_body(x_ref, w_ref, b_ref, o_ref):
    c = pl.program_id(0)
    bias = b_ref[pl.ds(c, 1), :]
    o_ref[...] = (
        jnp.dot(x_ref[...], w_ref[...], preferred_element_type=jnp.float32)
        + bias
    )


def _linear(xp, W, b2):
    return pl.pallas_call(
        _mm_body,
        grid=(NC, NNP // MMB),
        in_specs=[
            pl.BlockSpec((MMB, D), lambda c, k: (k, 0)),
            pl.BlockSpec((D, H), lambda c, k: (0, c)),
            pl.BlockSpec((NC, H), lambda c, k: (0, 0)),
        ],
        out_specs=pl.BlockSpec((MMB, H), lambda c, k: (c * (NNP // MMB) + k, 0)),
        out_shape=jax.ShapeDtypeStruct((NC * NNP, H), jnp.float32),
    )(xp, W, b2)


def _prep_body(srcs, dsts, eas,
               ew_out,
               deg,
               src2d, dst2d, ew2d, z1d, dinv_v,
               dsem):
    c = lax.axis_index("c")
    s = lax.axis_index("s")
    z16 = jnp.zeros((16,), jnp.float32)

    pltpu.sync_copy(srcs.at[s], src2d)
    pltpu.sync_copy(dsts.at[s], dst2d)
    pltpu.sync_copy(eas.at[s], ew2d)

    @pl.loop(0, RT // 16)
    def _(i):
        z1d[pl.ds(i * 16, 16)] = z16

    pltpu.sync_copy(z1d, deg.at[pl.ds(s * RT, RT)])
    plsc.subcore_barrier()

    def deg_wait(ch):
        pltpu.make_async_copy(eas.at[s, ch], ew2d.at[ch], dsem).wait()

    @pl.loop(0, NCHA)
    def _(ch):
        pltpu.async_copy(ew2d.at[ch], deg.at[dst2d.at[ch]], dsem, add=True)

        @pl.when(ch >= 8)
        def _():
            deg_wait(ch - 8)

    @pl.loop(NCHA - 8, NCHA)
    def _(ch):
        deg_wait(ch)

    plsc.subcore_barrier()

    pltpu.sync_copy(deg.at[pl.ds(s * RT, RT)], z1d)

    @pl.loop(0, RT // 16)
    def _(i):
        sl = pl.ds(i * 16, 16)
        v = z1d[sl]
        vm = jnp.maximum(v, 1e-12)
        bits = lax.bitcast_convert_type(vm, jnp.int32)
        y = lax.bitcast_convert_type(
            jnp.int32(0x5F3759DF) - (bits >> 1), jnp.float32
        )
        for _ in range(3):
            y = y * (1.5 - 0.5 * vm * y * y)
        z1d[sl] = jnp.where(v > 0.0, y, 0.0)

    pltpu.sync_copy(z1d, deg.at[pl.ds(s * RT, RT)])
    plsc.subcore_barrier()
    pltpu.sync_copy(deg, dinv_v)

    @pl.loop(0, NCHA)
    def _(ch):
        for k in range(CHA // 16):
            sl = pl.ds(k * 16, 16)
            dv_s = plsc.load_gather(dinv_v, [src2d[ch, sl]])
            dv_d = plsc.load_gather(dinv_v, [dst2d[ch, sl]])
            ew2d[ch, sl] = dv_s * ew2d[ch, sl] * dv_d

    @pl.when(c == 0)
    def _():
        pltpu.sync_copy(ew2d.at[pl.ds(0, 63)], ew_out.at[s, pl.ds(0, 63)])

    @pl.when(c == 1)
    def _():
        pltpu.sync_copy(ew2d.at[pl.ds(63, 62)], ew_out.at[s, pl.ds(63, 62)])


_prep_call = pl.kernel(
    _prep_body,
    out_type=[jax.ShapeDtypeStruct((NS, NCHA, CHA), jnp.float32)],
    mesh=plsc.VectorSubcoreMesh(
        core_axis_name="c", subcore_axis_name="s", num_cores=NC, num_subcores=NS
    ),
    compiler_params=pltpu.CompilerParams(
        needs_layout_passes=False, use_tc_tiling_on_sc=False
    ),
    scratch_types=[
        pltpu.VMEM_SHARED((NNP,), jnp.float32),
        pltpu.VMEM((NCHA, CHA), jnp.int32),
        pltpu.VMEM((NCHA, CHA), jnp.int32),
        pltpu.VMEM((NCHA, CHA), jnp.float32),
        pltpu.VMEM((RT,), jnp.float32),
        pltpu.VMEM((NNP,), jnp.float32),
        pltpu.SemaphoreType.DMA,
    ],
)


def _spmm_body(xsrc, srcs, dsts, ews, coefs,
               y1, y2, y3,
               acc,
               src2d, dst2d, ew2d, g0, g1, g2, g3, s0, s1, s2, s3, coefv,
               gs0, gs1, gs2, gs3, ss0, ss1, ss2, ss3, cs0, cs1, os0, os1):
    c = lax.axis_index("c")
    s = lax.axis_index("s")
    coff = c * NNP

    z16 = jnp.zeros((16,), jnp.float32)

    pltpu.sync_copy(srcs.at[s], src2d)
    pltpu.sync_copy(dsts.at[s], dst2d)
    pltpu.sync_copy(ews.at[s], ew2d)
    pltpu.sync_copy(coefs, coefv)

    @pl.loop(0, NCHB)
    def _(ch):
        src2d[ch, :] = src2d[ch, :] + coff

    levels = (
        (xsrc, None, y1, 0),
        (y1, xsrc, y2, 1),
        (y2, y1, y3, 2),
    )
    def dummy_wait(dst, sem):
        pltpu.make_async_copy(xsrc.at[pl.ds(0, CHB)], dst, sem).wait()

    for srcref, x2ref, yout, li in levels:
        @pl.loop(0, CHB)
        def _(r):
            for j in range(H // 16):
                s0[r, pl.ds(j * 16, 16)] = z16

        @pl.loop(0, NRC)
        def _(k):
            pltpu.sync_copy(s0, acc.at[pl.ds(s * RT + k * RC, RC)])

        plsc.subcore_barrier()

        quads = (
            (g0, s0, gs0, ss0), (g1, s1, gs1, ss1),
            (g2, s2, gs2, ss2), (g3, s3, gs3, ss3),
        )
        for b in range(4):
            pltpu.async_copy(srcref.at[src2d.at[b]], quads[b][0], quads[b][2])

        @pl.loop(0, NCHB)
        def _(ch):
            for b, (gb, sb, gsem, ssem) in enumerate(quads):
                @pl.when(ch % 4 == b)
                def _():
                    pltpu.make_async_copy(srcref.at[src2d.at[ch]], gb, gsem).wait()

                    @pl.when(ch >= 4)
                    def _():
                        dummy_wait(sb, ssem)

                    wv = ew2d[ch, :]
                    for e in range(CHB):
                        w = wv[e]
                        for j in range(H // 16):
                            sl = pl.ds(j * 16, 16)
                            sb[e, sl] = gb[e, sl] * w

                    @pl.when(ch + 4 < NCHB)
                    def _():
                        pltpu.async_copy(srcref.at[src2d.at[ch + 4]], gb, gsem)

                    pltpu.async_copy(sb, acc.at[dst2d.at[ch]], ssem, add=True)

        for b in range(4):
            dummy_wait(quads[b][1], quads[b][3])
        plsc.subcore_barrier()

        cv = coefv[...]
        cY = cv[3 * li + 0]
        cX2 = cv[3 * li + 2]

        @pl.loop(0, NRC)
        def _(k):
            base = s * RT + k * RC
            pltpu.sync_copy(acc.at[pl.ds(base, RC)], g0)
            if x2ref is not None:
                pltpu.sync_copy(x2ref.at[pl.ds(coff + base, RC)], s0)

            @pl.loop(0, RC)
            def _(r):
                for j in range(H // 16):
                    sl = pl.ds(j * 16, 16)
                    if x2ref is not None:
                        g0[r, sl] = cY * g0[r, sl] + cX2 * s0[r, sl]
                    else:
                        g0[r, sl] = cY * g0[r, sl]

            pltpu.sync_copy(g0, yout.at[pl.ds(coff + base, RC)])

        plsc.subcore_barrier()


_spmm_call = pl.kernel(
    _spmm_body,
    out_type=[jax.ShapeDtypeStruct((NC * NNP, H), jnp.float32)] * DEPTH,
    mesh=plsc.VectorSubcoreMesh(
        core_axis_name="c", subcore_axis_name="s", num_cores=NC, num_subcores=NS
    ),
    compiler_params=pltpu.CompilerParams(
        needs_layout_passes=False, use_tc_tiling_on_sc=False
    ),
    scratch_types=[
        pltpu.VMEM_SHARED((NNP, H), jnp.float32),
        pltpu.VMEM((NCHB, CHB), jnp.int32),
        pltpu.VMEM((NCHB, CHB), jnp.int32),
        pltpu.VMEM((NCHB, CHB), jnp.float32),
        pltpu.VMEM((CHB, H), jnp.float32),
        pltpu.VMEM((CHB, H), jnp.float32),
        pltpu.VMEM((CHB, H), jnp.float32),
        pltpu.VMEM((CHB, H), jnp.float32),
        pltpu.VMEM((CHB, H), jnp.float32),
        pltpu.VMEM((CHB, H), jnp.float32),
        pltpu.VMEM((CHB, H), jnp.float32),
        pltpu.VMEM((CHB, H), jnp.float32),
        pltpu.VMEM((16,), jnp.float32),
        pltpu.SemaphoreType.DMA,
        pltpu.SemaphoreType.DMA,
        pltpu.SemaphoreType.DMA,
        pltpu.SemaphoreType.DMA,
        pltpu.SemaphoreType.DMA,
        pltpu.SemaphoreType.DMA,
        pltpu.SemaphoreType.DMA,
        pltpu.SemaphoreType.DMA,
        pltpu.SemaphoreType.DMA,
        pltpu.SemaphoreType.DMA,
        pltpu.SemaphoreType.DMA,
        pltpu.SemaphoreType.DMA,
    ],
)


def _jacobi_coefs(alphas_param):
    alphas = jnp.tanh(alphas_param)
    a_, b_, l_, r_ = 1.0, 1.0, -1.0, 1.0
    cs = []
    coef1 = ((a_ - b_) / 2.0 - (a_ + b_ + 2.0) / 2.0 * (l_ + r_) / (r_ - l_))
    cs += [((a_ + b_ + 2.0) / (r_ - l_)) * alphas[0], coef1 * alphas[0],
           jnp.float32(0.0)]
    for L in range(2, DEPTH + 1):
        coef_l = 2.0 * L * (L + a_ + b_) * (2.0 * L - 2.0 + a_ + b_)
        coef_lm1_1 = (2.0 * L + a_ + b_ - 1.0) * (2.0 * L + a_ + b_) * (
            2.0 * L + a_ + b_ - 2.0)
        coef_lm1_2 = (2.0 * L + a_ + b_ - 1.0) * (a_ * a_ - b_ * b_)
        coef_lm2 = 2.0 * (L - 1.0 + a_) * (L - 1.0 + b_) * (2.0 * L + a_ + b_)
        tmp1 = alphas[L - 1] * (coef_lm1_1 / coef_l)
        tmp2 = alphas[L - 1] * (coef_lm1_2 / coef_l)
        tmp3 = alphas[L - 1] * alphas[L - 2] * (coef_lm2 / coef_l)
        tmp1_2 = tmp1 * (2.0 / (r_ - l_))
        tmp2_2 = tmp1 * ((r_ + l_) / (r_ - l_)) + tmp2
        cs += [tmp1_2, -tmp2_2, -tmp3]
    cs += [jnp.float32(0.0)] * (16 - len(cs))
    return jnp.stack(cs).astype(jnp.float32)


def kernel(x, edge_index, edge_attr, W, b, alphas_param):
    ei = edge_index.astype(jnp.int32)
    ea = edge_attr.astype(jnp.float32)
    srcs_a = ei[0].reshape(NS, NCHA, CHA)
    dsts_a = ei[1].reshape(NS, NCHA, CHA)
    eas_a = ea.reshape(NS, NCHA, CHA)
    srcs_b = ei[0].reshape(NS, NCHB, CHB)
    dsts_b = ei[1].reshape(NS, NCHB, CHB)
    xp = jnp.pad(x, ((0, NNP - NN), (0, 0)))
    b2 = b.reshape(NC, H)

    xsrc = _linear(xp, W, b2)
    coefs = _jacobi_coefs(alphas_param)
    (ew,) = _prep_call(srcs_a, dsts_a, eas_a)
    ews_b = ew.reshape(NS, NCHB, CHB)
    y1, y2, y3 = _spmm_call(xsrc, srcs_b, dsts_b, ews_b, coefs)

    def half(yv):
        return jnp.concatenate([yv[:NN], yv[NNP:NNP + NN]], axis=1)

    return jnp.stack([half(xsrc), half(y1), half(y2), half(y3)], axis=1)


# speedup vs baseline: 9.0285x; 1.0784x over previous
import jax
import jax.numpy as jnp
from jax import lax
from jax.experimental import pallas as pl
from jax.experimental.pallas import tpu as pltpu
from jax.experimental.pallas import tpu_sc as plsc

NN = 10000
NNP = 10240
NE = 160000
D = 256
H = 128
NC = 2
NS = 16
EPT = NE // NS
CHA = 80
NCHA = EPT // CHA
CHB = 16
NCHB = EPT // CHB
RT = NNP // NS
RC = 16
NRC = RT // RC
MMB = 640
DEPTH = 3


def _mm_body(x_ref, w_ref, b_ref, o_ref):
    c = pl.program_id(0)
    bias = b_ref[pl.ds(c, 1), :]
    o_ref[...] = (
        jnp.dot(x_ref[...], w_ref[...], preferred_element_type=jnp.float32)
        + bias
    )


def _linear(xp, W, b2):
    return pl.pallas_call(
        _mm_body,
        grid=(NC, NNP // MMB),
        in_specs=[
            pl.BlockSpec((MMB, D), lambda c, k: (k, 0)),
            pl.BlockSpec((D, H), lambda c, k: (0, c)),
            pl.BlockSpec((NC, H), lambda c, k: (0, 0)),
        ],
        out_specs=pl.BlockSpec((MMB, H), lambda c, k: (c * (NNP // MMB) + k, 0)),
        out_shape=jax.ShapeDtypeStruct((NC * NNP, H), jnp.float32),
    )(xp, W, b2)


def _prep_body(srcs, dsts, eas,
               ew_out,
               deg,
               src2d, dst2d, ew2d, z1d, dinv_v,
               dsem):
    c = lax.axis_index("c")
    s = lax.axis_index("s")
    z16 = jnp.zeros((16,), jnp.float32)

    pltpu.sync_copy(srcs.at[s], src2d)
    pltpu.sync_copy(dsts.at[s], dst2d)
    pltpu.sync_copy(eas.at[s], ew2d)

    @pl.loop(0, RT // 16)
    def _(i):
        z1d[pl.ds(i * 16, 16)] = z16

    pltpu.sync_copy(z1d, deg.at[pl.ds(s * RT, RT)])
    plsc.subcore_barrier()

    def deg_wait(ch):
        pltpu.make_async_copy(eas.at[s, ch], ew2d.at[ch], dsem).wait()

    @pl.loop(0, NCHA)
    def _(ch):
        pltpu.async_copy(ew2d.at[ch], deg.at[dst2d.at[ch]], dsem, add=True)

        @pl.when(ch >= 8)
        def _():
            deg_wait(ch - 8)

    @pl.loop(NCHA - 8, NCHA)
    def _(ch):
        deg_wait(ch)

    plsc.subcore_barrier()

    pltpu.sync_copy(deg.at[pl.ds(s * RT, RT)], z1d)

    @pl.loop(0, RT // 16)
    def _(i):
        sl = pl.ds(i * 16, 16)
        v = z1d[sl]
        vm = jnp.maximum(v, 1e-12)
        bits = lax.bitcast_convert_type(vm, jnp.int32)
        y = lax.bitcast_convert_type(
            jnp.int32(0x5F3759DF) - (bits >> 1), jnp.float32
        )
        for _ in range(3):
            y = y * (1.5 - 0.5 * vm * y * y)
        z1d[sl] = jnp.where(v > 0.0, y, 0.0)

    pltpu.sync_copy(z1d, deg.at[pl.ds(s * RT, RT)])
    plsc.subcore_barrier()
    pltpu.sync_copy(deg, dinv_v)

    @pl.loop(0, NCHA)
    def _(ch):
        for k in range(CHA // 16):
            sl = pl.ds(k * 16, 16)
            dv_s = plsc.load_gather(dinv_v, [src2d[ch, sl]])
            dv_d = plsc.load_gather(dinv_v, [dst2d[ch, sl]])
            ew2d[ch, sl] = dv_s * ew2d[ch, sl] * dv_d

    @pl.when(c == 0)
    def _():
        pltpu.sync_copy(ew2d.at[pl.ds(0, 63)], ew_out.at[s, pl.ds(0, 63)])

    @pl.when(c == 1)
    def _():
        pltpu.sync_copy(ew2d.at[pl.ds(63, 62)], ew_out.at[s, pl.ds(63, 62)])


_prep_call = pl.kernel(
    _prep_body,
    out_type=[jax.ShapeDtypeStruct((NS, NCHA, CHA), jnp.float32)],
    mesh=plsc.VectorSubcoreMesh(
        core_axis_name="c", subcore_axis_name="s", num_cores=NC, num_subcores=NS
    ),
    compiler_params=pltpu.CompilerParams(
        needs_layout_passes=False, use_tc_tiling_on_sc=False
    ),
    scratch_types=[
        pltpu.VMEM_SHARED((NNP,), jnp.float32),
        pltpu.VMEM((NCHA, CHA), jnp.int32),
        pltpu.VMEM((NCHA, CHA), jnp.int32),
        pltpu.VMEM((NCHA, CHA), jnp.float32),
        pltpu.VMEM((RT,), jnp.float32),
        pltpu.VMEM((NNP,), jnp.float32),
        pltpu.SemaphoreType.DMA,
    ],
)


def _spmm_body(xsrc, srcs, dsts, ews, coefs,
               y1, y2, y3,
               acc,
               src2d, dst2d, ew2d, g0, g1, g2, g3, s0, s1, s2, s3, coefv,
               gs0, gs1, gs2, gs3, ss0, ss1, ss2, ss3, cs0, cs1, os0, os1):
    c = lax.axis_index("c")
    s = lax.axis_index("s")
    coff = c * NNP

    z16 = jnp.zeros((16,), jnp.float32)

    pltpu.sync_copy(srcs.at[s], src2d)
    pltpu.sync_copy(dsts.at[s], dst2d)
    pltpu.sync_copy(ews.at[s], ew2d)
    pltpu.sync_copy(coefs, coefv)

    @pl.loop(0, NCHB)
    def _(ch):
        src2d[ch, :] = src2d[ch, :] + coff

    levels = (
        (xsrc, None, y1, 0),
        (y1, xsrc, y2, 1),
        (y2, y1, y3, 2),
    )
    def dummy_wait(dst, sem):
        pltpu.make_async_copy(xsrc.at[pl.ds(0, CHB)], dst, sem).wait()

    for srcref, x2ref, yout, li in levels:
        @pl.loop(0, CHB)
        def _(r):
            for j in range(H // 16):
                s0[r, pl.ds(j * 16, 16)] = z16

        @pl.loop(0, NRC)
        def _(k):
            pltpu.sync_copy(s0, acc.at[pl.ds(s * RT + k * RC, RC)])

        plsc.subcore_barrier()

        quads = (
            (g0, s0, gs0, ss0), (g1, s1, gs1, ss1),
            (g2, s2, gs2, ss2), (g3, s3, gs3, ss3),
        )
        for b in range(4):
            pltpu.async_copy(srcref.at[src2d.at[b]], quads[b][0], quads[b][2])

        @pl.loop(0, NCHB)
        def _(ch):
            for b, (gb, sb, gsem, ssem) in enumerate(quads):
                @pl.when(ch % 4 == b)
                def _():
                    pltpu.make_async_copy(srcref.at[src2d.at[ch]], gb, gsem).wait()

                    @pl.when(ch >= 4)
                    def _():
                        dummy_wait(sb, ssem)

                    wv = ew2d[ch, :]
                    for e in range(CHB):
                        w = wv[e]
                        for j in range(H // 16):
                            sl = pl.ds(j * 16, 16)
                            sb[e, sl] = gb[e, sl] * w

                    @pl.when(ch + 4 < NCHB)
                    def _():
                        pltpu.async_copy(srcref.at[src2d.at[ch + 4]], gb, gsem)

                    pltpu.async_copy(sb, acc.at[dst2d.at[ch]], ssem, add=True)

        for b in range(4):
            dummy_wait(quads[b][1], quads[b][3])
        plsc.subcore_barrier()

        cv = coefv[...]
        cY = cv[3 * li + 0]
        cX2 = cv[3 * li + 2]

        slots = ((g0, s0, cs0, os0), (g2, s2, cs1, os1))

        def x2_fetch(k, x2b, csem):
            base = s * RT + k * RC
            pltpu.async_copy(x2ref.at[pl.ds(coff + base, RC)], x2b, csem)

        def x2_wait(k, x2b, csem):
            base = s * RT + k * RC
            pltpu.make_async_copy(
                x2ref.at[pl.ds(coff + base, RC)], x2b, csem
            ).wait()

        if x2ref is not None:
            x2_fetch(0, slots[0][1], slots[0][2])
            x2_fetch(1, slots[1][1], slots[1][2])

        @pl.loop(0, NRC)
        def _(k):
            for b, (yb, x2b, csem, osem) in enumerate(slots):
                @pl.when(k % 2 == b)
                def _():
                    @pl.when(k >= 2)
                    def _():
                        dummy_wait(yb, osem)

                    if x2ref is not None:
                        x2_wait(k, x2b, csem)
                    base = s * RT + k * RC
                    pltpu.sync_copy(acc.at[pl.ds(base, RC)], yb)

                    @pl.loop(0, RC)
                    def _(r):
                        for j in range(H // 16):
                            sl = pl.ds(j * 16, 16)
                            if x2ref is not None:
                                yb[r, sl] = cY * yb[r, sl] + cX2 * x2b[r, sl]
                            else:
                                yb[r, sl] = cY * yb[r, sl]

                    pltpu.async_copy(yb, yout.at[pl.ds(coff + base, RC)], osem)

                    if x2ref is not None:
                        @pl.when(k + 2 < NRC)
                        def _():
                            x2_fetch(k + 2, x2b, csem)

        dummy_wait(g0, os0)
        dummy_wait(g2, os1)
        plsc.subcore_barrier()


_spmm_call = pl.kernel(
    _spmm_body,
    out_type=[jax.ShapeDtypeStruct((NC * NNP, H), jnp.float32)] * DEPTH,
    mesh=plsc.VectorSubcoreMesh(
        core_axis_name="c", subcore_axis_name="s", num_cores=NC, num_subcores=NS
    ),
    compiler_params=pltpu.CompilerParams(
        needs_layout_passes=False, use_tc_tiling_on_sc=False
    ),
    scratch_types=[
        pltpu.VMEM_SHARED((NNP, H), jnp.float32),
        pltpu.VMEM((NCHB, CHB), jnp.int32),
        pltpu.VMEM((NCHB, CHB), jnp.int32),
        pltpu.VMEM((NCHB, CHB), jnp.float32),
        pltpu.VMEM((CHB, H), jnp.float32),
        pltpu.VMEM((CHB, H), jnp.float32),
        pltpu.VMEM((CHB, H), jnp.float32),
        pltpu.VMEM((CHB, H), jnp.float32),
        pltpu.VMEM((CHB, H), jnp.float32),
        pltpu.VMEM((CHB, H), jnp.float32),
        pltpu.VMEM((CHB, H), jnp.float32),
        pltpu.VMEM((CHB, H), jnp.float32),
        pltpu.VMEM((16,), jnp.float32),
        pltpu.SemaphoreType.DMA,
        pltpu.SemaphoreType.DMA,
        pltpu.SemaphoreType.DMA,
        pltpu.SemaphoreType.DMA,
        pltpu.SemaphoreType.DMA,
        pltpu.SemaphoreType.DMA,
        pltpu.SemaphoreType.DMA,
        pltpu.SemaphoreType.DMA,
        pltpu.SemaphoreType.DMA,
        pltpu.SemaphoreType.DMA,
        pltpu.SemaphoreType.DMA,
        pltpu.SemaphoreType.DMA,
    ],
)


def _jacobi_coefs(alphas_param):
    alphas = jnp.tanh(alphas_param)
    a_, b_, l_, r_ = 1.0, 1.0, -1.0, 1.0
    cs = []
    coef1 = ((a_ - b_) / 2.0 - (a_ + b_ + 2.0) / 2.0 * (l_ + r_) / (r_ - l_))
    cs += [((a_ + b_ + 2.0) / (r_ - l_)) * alphas[0], coef1 * alphas[0],
           jnp.float32(0.0)]
    for L in range(2, DEPTH + 1):
        coef_l = 2.0 * L * (L + a_ + b_) * (2.0 * L - 2.0 + a_ + b_)
        coef_lm1_1 = (2.0 * L + a_ + b_ - 1.0) * (2.0 * L + a_ + b_) * (
            2.0 * L + a_ + b_ - 2.0)
        coef_lm1_2 = (2.0 * L + a_ + b_ - 1.0) * (a_ * a_ - b_ * b_)
        coef_lm2 = 2.0 * (L - 1.0 + a_) * (L - 1.0 + b_) * (2.0 * L + a_ + b_)
        tmp1 = alphas[L - 1] * (coef_lm1_1 / coef_l)
        tmp2 = alphas[L - 1] * (coef_lm1_2 / coef_l)
        tmp3 = alphas[L - 1] * alphas[L - 2] * (coef_lm2 / coef_l)
        tmp1_2 = tmp1 * (2.0 / (r_ - l_))
        tmp2_2 = tmp1 * ((r_ + l_) / (r_ - l_)) + tmp2
        cs += [tmp1_2, -tmp2_2, -tmp3]
    cs += [jnp.float32(0.0)] * (16 - len(cs))
    return jnp.stack(cs).astype(jnp.float32)


def kernel(x, edge_index, edge_attr, W, b, alphas_param):
    ei = edge_index.astype(jnp.int32)
    ea = edge_attr.astype(jnp.float32)
    srcs_a = ei[0].reshape(NS, NCHA, CHA)
    dsts_a = ei[1].reshape(NS, NCHA, CHA)
    eas_a = ea.reshape(NS, NCHA, CHA)
    srcs_b = ei[0].reshape(NS, NCHB, CHB)
    dsts_b = ei[1].reshape(NS, NCHB, CHB)
    xp = jnp.pad(x, ((0, NNP - NN), (0, 0)))
    b2 = b.reshape(NC, H)

    xsrc = _linear(xp, W, b2)
    coefs = _jacobi_coefs(alphas_param)
    (ew,) = _prep_call(srcs_a, dsts_a, eas_a)
    ews_b = ew.reshape(NS, NCHB, CHB)
    y1, y2, y3 = _spmm_call(xsrc, srcs_b, dsts_b, ews_b, coefs)

    def half(yv):
        return jnp.concatenate([yv[:NN], yv[NNP:NNP + NN]], axis=1)

    return jnp.stack([half(xsrc), half(y1), half(y2), half(y3)], axis=1)
